# baseline (speedup 1.0000x reference)
import jax
import jax.numpy as jnp
from jax.experimental import pallas as pl
from jax.experimental.pallas import tpu as pltpu

TOP_K = 20
BN_EPS = 1e-5


_DEPTH = 6


def _lane_fold(x, op):
    while x.shape[1] > 128:
        h = x.shape[1] // 2
        x = op(x[:, :h], x[:, h:])
    return x


def _count_rounds(vals, cnts, nd):
    bt = vals[0].shape[0]
    zero = jnp.zeros((bt, 1), jnp.float32)
    rc, thr, nkeep = zero, zero - 1.0, zero
    found = zero > 1.0
    suspect = found
    for r in range(TOP_K):
        mm = vals[0]
        for d in range(1, nd):
            mm = jnp.maximum(mm, vals[d])
        m = jnp.max(mm, axis=1, keepdims=True)
        eqs = [vals[d] == m for d in range(nd)]
        cs = jnp.where(eqs[0], cnts[0], 0.0)
        for d in range(1, nd):
            cs = cs + jnp.where(eqs[d], cnts[d], 0.0)
        c = jnp.sum(cs, axis=1, keepdims=True)
        found_now = jnp.logical_and(jnp.logical_not(found), rc + c >= TOP_K)
        thr = jnp.where(found_now, m, thr)
        nkeep = jnp.where(found_now, TOP_K - rc, nkeep)
        found = jnp.logical_or(found, found_now)
        rc = rc + c
        if nd > 1:
            deep = jnp.max(jnp.where(eqs[nd - 1], 1.0, 0.0), axis=1,
                           keepdims=True) > 0.0
            suspect = jnp.logical_or(suspect,
                                     jnp.logical_and(deep,
                                                     jnp.logical_not(found)))
        if r < TOP_K - 1:
            for d in range(nd):
                vals[d] = jnp.where(eqs[d], -1.0, vals[d])
    return thr, nkeep, suspect


def _thr_body(sae_ref, thr_ref, idx_ref, work, vals_s, cnts_s, thr_s, nk_s,
              rc_s, fd_s):
    bt, f = sae_ref.shape
    work[...] = sae_ref[...]

    def _depth_body(d, carry):
        w = work[...]
        m = _lane_fold(w, jnp.maximum)
        eq = w == jnp.tile(m, (1, f // 128))
        cnt = _lane_fold(jnp.where(eq, 1.0, 0.0), jnp.add)
        vals_s[d] = m
        cnts_s[d] = jnp.where(m >= 0.0, cnt, 0.0)
        work[...] = jnp.where(eq, -1.0, w)
        return carry

    jax.lax.fori_loop(0, _DEPTH, _depth_body, 0)
    vals = [vals_s[d] for d in range(_DEPTH)]
    cnts = [cnts_s[d] for d in range(_DEPTH)]
    thr, nkeep, suspect = _count_rounds(vals, cnts, _DEPTH)
    thr_s[...] = thr
    nk_s[...] = nkeep

    @pl.when(jnp.max(jnp.where(suspect, 1.0, 0.0)) > 0.0)
    def _():
        work[...] = sae_ref[...]
        rc_s[...] = jnp.zeros((bt, 1), jnp.float32)
        fd_s[...] = jnp.zeros((bt, 1), jnp.float32)

        def _bf_round(_r, carry):
            wv = work[...]
            m = jnp.max(wv, axis=1, keepdims=True)
            eq = wv == m
            c = jnp.sum(jnp.where(eq, 1.0, 0.0), axis=1, keepdims=True)
            rc = rc_s[...]
            fn = jnp.where(jnp.logical_and(fd_s[...] < 0.5, rc + c >= TOP_K),
                           1.0, 0.0)
            upd = jnp.where(suspect, fn, 0.0) > 0.5
            thr_s[...] = jnp.where(upd, m, thr_s[...])
            nk_s[...] = jnp.where(upd, TOP_K - rc, nk_s[...])
            fd_s[...] = jnp.maximum(fd_s[...], fn)
            rc_s[...] = rc + c
            work[...] = jnp.where(eq, -1.0, wv)
            return carry

        jax.lax.fori_loop(0, TOP_K, _bf_round, 0)

    iota_f = jax.lax.broadcasted_iota(jnp.int32, (bt, f), 1).astype(jnp.float32)
    thr_v = thr_s[...]
    work[...] = jnp.where(sae_ref[...] == thr_v, iota_f, jnp.inf)
    nk_s[...] = nk_s[...] - 1.0

    def _tie_round(_i, carry):
        @pl.when(jnp.max(nk_s[...]) > 0.0)
        def _():
            wv = work[...]
            mn = jnp.min(wv, axis=1, keepdims=True)
            rem = nk_s[...] > 0.0
            work[...] = jnp.where(jnp.logical_and(rem, wv == mn), jnp.inf, wv)
            nk_s[...] = jnp.where(rem, nk_s[...] - 1.0, nk_s[...])
        return carry

    jax.lax.fori_loop(0, TOP_K - 1, _tie_round, 0)
    thr_ref[...] = thr_v
    idx_ref[...] = jnp.min(work[...], axis=1, keepdims=True).astype(jnp.int32)


def _topk_thresholds(sae, bt):
    b, f = sae.shape
    return pl.pallas_call(
        _thr_body,
        grid=(b // bt,),
        in_specs=[pl.BlockSpec((bt, f), lambda i: (i, 0))],
        out_specs=[pl.BlockSpec((bt, 1), lambda i: (i, 0)),
                   pl.BlockSpec((bt, 1), lambda i: (i, 0))],
        out_shape=[jax.ShapeDtypeStruct((b, 1), jnp.float32),
                   jax.ShapeDtypeStruct((b, 1), jnp.int32)],
        scratch_shapes=[
            pltpu.VMEM((bt, f), jnp.float32),
            pltpu.VMEM((_DEPTH, bt, 128), jnp.float32),
            pltpu.VMEM((_DEPTH, bt, 128), jnp.float32),
            pltpu.VMEM((bt, 1), jnp.float32),
            pltpu.VMEM((bt, 1), jnp.float32),
            pltpu.VMEM((bt, 1), jnp.float32),
            pltpu.VMEM((bt, 1), jnp.float32),
        ],
        compiler_params=pltpu.CompilerParams(
            dimension_semantics=("parallel",),
            vmem_limit_bytes=48 * 1024 * 1024,
        ),
        name="topk_thr",
    )(sae)


def _bi_body(sae_ref, thr_ref, idx_ref, emb_ref, bi_ref, s_ref, q_ref,
             acc1, acc2, bk):
    k = pl.program_id(1)
    nk = pl.num_programs(1)

    @pl.when(k == 0)
    def _():
        acc1[...] = jnp.zeros_like(acc1)
        acc2[...] = jnp.zeros_like(acc2)

    blk = sae_ref[...]
    thr = thr_ref[...]
    g_iota = jax.lax.broadcasted_iota(jnp.int32, blk.shape, 1) + k * bk
    keep = (blk > thr) | ((blk == thr) & (g_iota <= idx_ref[...]))
    x = jnp.where(keep, blk, 0.0)
    e = emb_ref[...]
    acc1[...] += jnp.dot(x.astype(jnp.bfloat16), e.astype(jnp.bfloat16),
                         preferred_element_type=jnp.float32)
    acc2[...] += jnp.dot((x * x).astype(jnp.bfloat16),
                         (e * e).astype(jnp.bfloat16),
                         preferred_element_type=jnp.float32)

    @pl.when(k == nk - 1)
    def _():
        s = acc1[...]
        bi = 0.5 * (s * s - acc2[...])
        bi_ref[...] = bi
        s_ref[...] = jnp.sum(bi, axis=0, keepdims=True)[None]
        q_ref[...] = jnp.sum(bi * bi, axis=0, keepdims=True)[None]


def _bi_interaction(sae, thr, idx, emb, bb, bk):
    import functools
    b, f = sae.shape
    d = emb.shape[1]
    nb, nk = b // bb, f // bk
    return pl.pallas_call(
        functools.partial(_bi_body, bk=bk),
        grid=(nb, nk),
        in_specs=[
            pl.BlockSpec((bb, bk), lambda i, k: (i, k)),
            pl.BlockSpec((bb, 1), lambda i, k: (i, 0)),
            pl.BlockSpec((bb, 1), lambda i, k: (i, 0)),
            pl.BlockSpec((bk, d), lambda i, k: (k, 0)),
        ],
        out_specs=[
            pl.BlockSpec((bb, d), lambda i, k: (i, 0)),
            pl.BlockSpec((1, 1, d), lambda i, k: (i, 0, 0)),
            pl.BlockSpec((1, 1, d), lambda i, k: (i, 0, 0)),
        ],
        out_shape=[
            jax.ShapeDtypeStruct((b, d), jnp.float32),
            jax.ShapeDtypeStruct((nb, 1, d), jnp.float32),
            jax.ShapeDtypeStruct((nb, 1, d), jnp.float32),
        ],
        scratch_shapes=[
            pltpu.VMEM((bb, d), jnp.float32),
            pltpu.VMEM((bb, d), jnp.float32),
        ],
        compiler_params=pltpu.CompilerParams(
            dimension_semantics=("parallel", "arbitrary"),
            vmem_limit_bytes=48 * 1024 * 1024,
        ),
        name="bi_pool",
    )(sae, thr, idx, emb)


def _mlp1_body(bi_ref, s1_ref, q1_ref, w1t_ref, b1_ref, g1_ref, be1_ref,
               h_ref, hs_ref, hq_ref, nrows):
    mu = jnp.sum(s1_ref[...], axis=(0, 1)) / nrows
    var = jnp.sum(q1_ref[...], axis=(0, 1)) / nrows - mu * mu
    a1 = g1_ref[0] * jax.lax.rsqrt(var + BN_EPS)
    c1 = be1_ref[0] - mu * a1
    bi_n = bi_ref[...] * a1[None, :] + c1[None, :]
    h = jnp.dot(bi_n.astype(jnp.bfloat16), w1t_ref[...].astype(jnp.bfloat16),
                preferred_element_type=jnp.float32) + b1_ref[...]
    h_ref[...] = h
    hs_ref[...] = jnp.sum(h, axis=0, keepdims=True)[None]
    hq_ref[...] = jnp.sum(h * h, axis=0, keepdims=True)[None]


def _mlp1(bi, s1, q1, w1t, b1, g1, be1, bb):
    b, d = bi.shape
    nb = b // bb
    import functools
    return pl.pallas_call(
        functools.partial(_mlp1_body, nrows=float(b)),
        grid=(nb,),
        in_specs=[
            pl.BlockSpec((bb, d), lambda i: (i, 0)),
            pl.BlockSpec(s1.shape, lambda i: (0, 0, 0)),
            pl.BlockSpec(q1.shape, lambda i: (0, 0, 0)),
            pl.BlockSpec((d, d), lambda i: (0, 0)),
            pl.BlockSpec((1, d), lambda i: (0, 0)),
            pl.BlockSpec((1, d), lambda i: (0, 0)),
            pl.BlockSpec((1, d), lambda i: (0, 0)),
        ],
        out_specs=[
            pl.BlockSpec((bb, d), lambda i: (i, 0)),
            pl.BlockSpec((1, 1, d), lambda i: (i, 0, 0)),
            pl.BlockSpec((1, 1, d), lambda i: (i, 0, 0)),
        ],
        out_shape=[
            jax.ShapeDtypeStruct((b, d), jnp.float32),
            jax.ShapeDtypeStruct((nb, 1, d), jnp.float32),
            jax.ShapeDtypeStruct((nb, 1, d), jnp.float32),
        ],
        compiler_params=pltpu.CompilerParams(
            dimension_semantics=("parallel",),
            vmem_limit_bytes=40 * 1024 * 1024,
        ),
        name="mlp1_bn",
    )(bi, s1, q1, w1t, b1, g1, be1)


def _final_body(sae_ref, w_ref, h_ref, hs_ref, hq_ref, w2_ref, g2_ref,
                be2_ref, b2_ref, lb_ref, gb_ref,
                out_ref, lin_ref, int_ref, acc, g_buf, nrows):
    o = pl.program_id(1)
    k = pl.program_id(2)
    nk = pl.num_programs(2)

    @pl.when(k == 0)
    def _():
        acc[...] = jnp.zeros_like(acc)

    @pl.when((o == 0) & (k == 0))
    def _():
        mu = jnp.sum(hs_ref[...], axis=(0, 1)) / nrows
        var = jnp.sum(hq_ref[...], axis=(0, 1)) / nrows - mu * mu
        a2 = g2_ref[0] * jax.lax.rsqrt(var + BN_EPS)
        c2 = be2_ref[0] - mu * a2
        g = jnp.maximum(h_ref[...] * a2[None, :] + c2[None, :], 0.0)
        g_buf[...] = g.astype(jnp.bfloat16)

    acc[...] += jax.lax.dot_general(
        sae_ref[...].astype(jnp.bfloat16), w_ref[...].astype(jnp.bfloat16),
        (((1,), (1,)), ((), ())), preferred_element_type=jnp.float32)

    @pl.when(k == nk - 1)
    def _():
        inter = jax.lax.dot_general(
            g_buf[...], w2_ref[...].astype(jnp.bfloat16),
            (((1,), (1,)), ((), ())),
            preferred_element_type=jnp.float32) + b2_ref[...]
        lin = acc[...] + lb_ref[...]
        lin_ref[...] = lin
        int_ref[...] = inter
        out_ref[...] = gb_ref[...] + lin + inter


def _final(sae, lin_w, h, hs, hq, w2, g2, be2, b2, lb, gb, bb, bo, bk):
    b, f = sae.shape
    o = lin_w.shape[0]
    d = h.shape[1]
    nb, no, nk = b // bb, o // bo, f // bk
    import functools
    out_shape = jax.ShapeDtypeStruct((b, o), jnp.float32)
    return pl.pallas_call(
        functools.partial(_final_body, nrows=float(b)),
        grid=(nb, no, nk),
        in_specs=[
            pl.BlockSpec((bb, bk), lambda i, j, k: (i, k)),
            pl.BlockSpec((bo, bk), lambda i, j, k: (j, k)),
            pl.BlockSpec((bb, d), lambda i, j, k: (i, 0)),
            pl.BlockSpec(hs.shape, lambda i, j, k: (0, 0, 0)),
            pl.BlockSpec(hq.shape, lambda i, j, k: (0, 0, 0)),
            pl.BlockSpec((bo, d), lambda i, j, k: (j, 0)),
            pl.BlockSpec((1, d), lambda i, j, k: (0, 0)),
            pl.BlockSpec((1, d), lambda i, j, k: (0, 0)),
            pl.BlockSpec((1, bo), lambda i, j, k: (0, j)),
            pl.BlockSpec((1, bo), lambda i, j, k: (0, j)),
            pl.BlockSpec((1, bo), lambda i, j, k: (0, j)),
        ],
        out_specs=[
            pl.BlockSpec((bb, bo), lambda i, j, k: (i, j)),
            pl.BlockSpec((bb, bo), lambda i, j, k: (i, j)),
            pl.BlockSpec((bb, bo), lambda i, j, k: (i, j)),
        ],
        out_shape=[out_shape, out_shape, out_shape],
        scratch_shapes=[
            pltpu.VMEM((bb, bo), jnp.float32),
            pltpu.VMEM((bb, d), jnp.bfloat16),
        ],
        compiler_params=pltpu.CompilerParams(
            dimension_semantics=("parallel", "arbitrary", "arbitrary"),
            vmem_limit_bytes=56 * 1024 * 1024,
        ),
        name="linear_mlp2_fused",
    )(sae, lin_w, h, hs, hq, w2, g2, be2, b2, lb, gb)


def kernel(sae_features, emb, lin_w, lin_b, global_bias, bn1_gamma, bn1_beta,
           mlp_w1, mlp_b1, bn2_gamma, bn2_beta, mlp_w2, mlp_b2):
    b, f = sae_features.shape
    d = emb.shape[1]

    thr, idx = _topk_thresholds(sae_features, bt=64)
    bi, s1, q1 = _bi_interaction(sae_features, thr, idx, emb, bb=1024, bk=2048)
    h, hs, hq = _mlp1(bi, s1, q1, mlp_w1.T, mlp_b1.reshape(1, d),
                      bn1_gamma.reshape(1, d), bn1_beta.reshape(1, d), bb=256)
    out, lin, inter = _final(
        sae_features, lin_w, h, hs, hq, mlp_w2,
        bn2_gamma.reshape(1, d), bn2_beta.reshape(1, d),
        mlp_b2.reshape(1, -1), lin_b.reshape(1, -1), global_bias.reshape(1, -1),
        bb=1024, bo=1024, bk=1024)
    return out, lin, inter


# speedup vs baseline: 1.0538x; 1.0538x over previous
import jax
import jax.numpy as jnp
from jax.experimental import pallas as pl
from jax.experimental.pallas import tpu as pltpu

TOP_K = 20
BN_EPS = 1e-5


_DEPTH = 6


def _lane_fold(x, op):
    while x.shape[1] > 128:
        h = x.shape[1] // 2
        x = op(x[:, :h], x[:, h:])
    return x


def _count_rounds(vals, cnts, nd):
    bt = vals[0].shape[0]
    zero = jnp.zeros((bt, 1), jnp.float32)
    rc, thr, nkeep = zero, zero - 1.0, zero
    found = zero > 1.0
    suspect = found
    for r in range(TOP_K):
        mm = vals[0]
        for d in range(1, nd):
            mm = jnp.maximum(mm, vals[d])
        m = jnp.max(mm, axis=1, keepdims=True)
        eqs = [vals[d] == m for d in range(nd)]
        cs = jnp.where(eqs[0], cnts[0], 0.0)
        for d in range(1, nd):
            cs = cs + jnp.where(eqs[d], cnts[d], 0.0)
        c = jnp.sum(cs, axis=1, keepdims=True)
        found_now = jnp.logical_and(jnp.logical_not(found), rc + c >= TOP_K)
        thr = jnp.where(found_now, m, thr)
        nkeep = jnp.where(found_now, TOP_K - rc, nkeep)
        found = jnp.logical_or(found, found_now)
        rc = rc + c
        if nd > 1:
            deep = jnp.max(jnp.where(eqs[nd - 1], 1.0, 0.0), axis=1,
                           keepdims=True) > 0.0
            suspect = jnp.logical_or(suspect,
                                     jnp.logical_and(deep,
                                                     jnp.logical_not(found)))
        if r < TOP_K - 1:
            for d in range(nd):
                vals[d] = jnp.where(eqs[d], -1.0, vals[d])
    return thr, nkeep, suspect


def _thr_body(sae_ref, thr_ref, idx_ref, work, vals_s, cnts_s, thr_s, nk_s,
              rc_s, fd_s):
    bt, f = sae_ref.shape
    work[...] = sae_ref[...]

    def _depth_body(d, carry):
        w = work[...]
        m = _lane_fold(w, jnp.maximum)
        eq = w == jnp.tile(m, (1, f // 128))
        cnt = _lane_fold(jnp.where(eq, 1.0, 0.0), jnp.add)
        vals_s[d] = m
        cnts_s[d] = jnp.where(m >= 0.0, cnt, 0.0)
        work[...] = jnp.where(eq, -1.0, w)
        return carry

    jax.lax.fori_loop(0, _DEPTH, _depth_body, 0)
    vals = [vals_s[d] for d in range(_DEPTH)]
    cnts = [cnts_s[d] for d in range(_DEPTH)]
    thr, nkeep, suspect = _count_rounds(vals, cnts, _DEPTH)
    thr_s[...] = thr
    nk_s[...] = nkeep

    @pl.when(jnp.max(jnp.where(suspect, 1.0, 0.0)) > 0.0)
    def _():
        work[...] = sae_ref[...]
        rc_s[...] = jnp.zeros((bt, 1), jnp.float32)
        fd_s[...] = jnp.zeros((bt, 1), jnp.float32)

        def _bf_round(_r, carry):
            wv = work[...]
            m = jnp.max(wv, axis=1, keepdims=True)
            eq = wv == m
            c = jnp.sum(jnp.where(eq, 1.0, 0.0), axis=1, keepdims=True)
            rc = rc_s[...]
            fn = jnp.where(jnp.logical_and(fd_s[...] < 0.5, rc + c >= TOP_K),
                           1.0, 0.0)
            upd = jnp.where(suspect, fn, 0.0) > 0.5
            thr_s[...] = jnp.where(upd, m, thr_s[...])
            nk_s[...] = jnp.where(upd, TOP_K - rc, nk_s[...])
            fd_s[...] = jnp.maximum(fd_s[...], fn)
            rc_s[...] = rc + c
            work[...] = jnp.where(eq, -1.0, wv)
            return carry

        jax.lax.fori_loop(0, TOP_K, _bf_round, 0)

    iota_f = jax.lax.broadcasted_iota(jnp.int32, (bt, f), 1).astype(jnp.float32)
    thr_v = thr_s[...]
    work[...] = jnp.where(sae_ref[...] == thr_v, iota_f, jnp.inf)
    nk_s[...] = nk_s[...] - 1.0

    def _tie_round(_i, carry):
        @pl.when(jnp.max(nk_s[...]) > 0.0)
        def _():
            wv = work[...]
            mn = jnp.min(wv, axis=1, keepdims=True)
            rem = nk_s[...] > 0.0
            work[...] = jnp.where(jnp.logical_and(rem, wv == mn), jnp.inf, wv)
            nk_s[...] = jnp.where(rem, nk_s[...] - 1.0, nk_s[...])
        return carry

    jax.lax.fori_loop(0, TOP_K - 1, _tie_round, 0)
    thr_ref[...] = thr_v
    idx_ref[...] = jnp.min(work[...], axis=1, keepdims=True).astype(jnp.int32)


def _topk_thresholds(sae, bt):
    b, f = sae.shape
    return pl.pallas_call(
        _thr_body,
        grid=(b // bt,),
        in_specs=[pl.BlockSpec((bt, f), lambda i: (i, 0))],
        out_specs=[pl.BlockSpec((bt, 1), lambda i: (i, 0)),
                   pl.BlockSpec((bt, 1), lambda i: (i, 0))],
        out_shape=[jax.ShapeDtypeStruct((b, 1), jnp.float32),
                   jax.ShapeDtypeStruct((b, 1), jnp.int32)],
        scratch_shapes=[
            pltpu.VMEM((bt, f), jnp.float32),
            pltpu.VMEM((_DEPTH, bt, 128), jnp.float32),
            pltpu.VMEM((_DEPTH, bt, 128), jnp.float32),
            pltpu.VMEM((bt, 1), jnp.float32),
            pltpu.VMEM((bt, 1), jnp.float32),
            pltpu.VMEM((bt, 1), jnp.float32),
            pltpu.VMEM((bt, 1), jnp.float32),
        ],
        compiler_params=pltpu.CompilerParams(
            dimension_semantics=("parallel",),
            vmem_limit_bytes=48 * 1024 * 1024,
        ),
        name="topk_thr",
    )(sae)


def _bi_body(sae_ref, thr_ref, idx_ref, emb_ref, bi_ref, s_ref, q_ref,
             acc1, acc2, bk):
    k = pl.program_id(1)
    nk = pl.num_programs(1)

    @pl.when(k == 0)
    def _():
        acc1[...] = jnp.zeros_like(acc1)
        acc2[...] = jnp.zeros_like(acc2)

    blk = sae_ref[...]
    thr = thr_ref[...]
    g_iota = jax.lax.broadcasted_iota(jnp.int32, blk.shape, 1) + k * bk
    keep = (blk > thr) | ((blk == thr) & (g_iota <= idx_ref[...]))
    x = jnp.where(keep, blk, 0.0)
    e = emb_ref[...]
    acc1[...] += jnp.dot(x.astype(jnp.bfloat16), e.astype(jnp.bfloat16),
                         preferred_element_type=jnp.float32)
    acc2[...] += jnp.dot((x * x).astype(jnp.bfloat16),
                         (e * e).astype(jnp.bfloat16),
                         preferred_element_type=jnp.float32)

    @pl.when(k == nk - 1)
    def _():
        s = acc1[...]
        bi = 0.5 * (s * s - acc2[...])
        bi_ref[...] = bi
        s_ref[...] = jnp.sum(bi, axis=0, keepdims=True)[None]
        q_ref[...] = jnp.sum(bi * bi, axis=0, keepdims=True)[None]


def _bi_interaction(sae, thr, idx, emb, bb, bk):
    import functools
    b, f = sae.shape
    d = emb.shape[1]
    nb, nk = b // bb, f // bk
    return pl.pallas_call(
        functools.partial(_bi_body, bk=bk),
        grid=(nb, nk),
        in_specs=[
            pl.BlockSpec((bb, bk), lambda i, k: (i, k)),
            pl.BlockSpec((bb, 1), lambda i, k: (i, 0)),
            pl.BlockSpec((bb, 1), lambda i, k: (i, 0)),
            pl.BlockSpec((bk, d), lambda i, k: (k, 0)),
        ],
        out_specs=[
            pl.BlockSpec((bb, d), lambda i, k: (i, 0)),
            pl.BlockSpec((1, 1, d), lambda i, k: (i, 0, 0)),
            pl.BlockSpec((1, 1, d), lambda i, k: (i, 0, 0)),
        ],
        out_shape=[
            jax.ShapeDtypeStruct((b, d), jnp.float32),
            jax.ShapeDtypeStruct((nb, 1, d), jnp.float32),
            jax.ShapeDtypeStruct((nb, 1, d), jnp.float32),
        ],
        scratch_shapes=[
            pltpu.VMEM((bb, d), jnp.float32),
            pltpu.VMEM((bb, d), jnp.float32),
        ],
        compiler_params=pltpu.CompilerParams(
            dimension_semantics=("parallel", "arbitrary"),
            vmem_limit_bytes=48 * 1024 * 1024,
        ),
        name="bi_pool",
    )(sae, thr, idx, emb)


def _mlp1_body(bi_ref, s1_ref, q1_ref, w1t_ref, b1_ref, g1_ref, be1_ref,
               h_ref, hs_ref, hq_ref, nrows):
    mu = jnp.sum(s1_ref[...], axis=(0, 1)) / nrows
    var = jnp.sum(q1_ref[...], axis=(0, 1)) / nrows - mu * mu
    a1 = g1_ref[0] * jax.lax.rsqrt(var + BN_EPS)
    c1 = be1_ref[0] - mu * a1
    bi_n = bi_ref[...] * a1[None, :] + c1[None, :]
    h = jnp.dot(bi_n.astype(jnp.bfloat16), w1t_ref[...].astype(jnp.bfloat16),
                preferred_element_type=jnp.float32) + b1_ref[...]
    h_ref[...] = h
    hs_ref[...] = jnp.sum(h, axis=0, keepdims=True)[None]
    hq_ref[...] = jnp.sum(h * h, axis=0, keepdims=True)[None]


def _mlp1(bi, s1, q1, w1t, b1, g1, be1, bb):
    b, d = bi.shape
    nb = b // bb
    import functools
    return pl.pallas_call(
        functools.partial(_mlp1_body, nrows=float(b)),
        grid=(nb,),
        in_specs=[
            pl.BlockSpec((bb, d), lambda i: (i, 0)),
            pl.BlockSpec(s1.shape, lambda i: (0, 0, 0)),
            pl.BlockSpec(q1.shape, lambda i: (0, 0, 0)),
            pl.BlockSpec((d, d), lambda i: (0, 0)),
            pl.BlockSpec((1, d), lambda i: (0, 0)),
            pl.BlockSpec((1, d), lambda i: (0, 0)),
            pl.BlockSpec((1, d), lambda i: (0, 0)),
        ],
        out_specs=[
            pl.BlockSpec((bb, d), lambda i: (i, 0)),
            pl.BlockSpec((1, 1, d), lambda i: (i, 0, 0)),
            pl.BlockSpec((1, 1, d), lambda i: (i, 0, 0)),
        ],
        out_shape=[
            jax.ShapeDtypeStruct((b, d), jnp.float32),
            jax.ShapeDtypeStruct((nb, 1, d), jnp.float32),
            jax.ShapeDtypeStruct((nb, 1, d), jnp.float32),
        ],
        compiler_params=pltpu.CompilerParams(
            dimension_semantics=("parallel",),
            vmem_limit_bytes=40 * 1024 * 1024,
        ),
        name="mlp1_bn",
    )(bi, s1, q1, w1t, b1, g1, be1)


def _final_body(sae_ref, w_ref, h_ref, hs_ref, hq_ref, w2_ref, g2_ref,
                be2_ref, b2_ref, lb_ref, gb_ref,
                out_ref, lin_ref, int_ref, acc, g_buf, nrows):
    o = pl.program_id(1)
    k = pl.program_id(2)
    nk = pl.num_programs(2)

    @pl.when(k == 0)
    def _():
        acc[...] = jnp.zeros_like(acc)

    @pl.when((o == 0) & (k == 0))
    def _():
        mu = jnp.sum(hs_ref[...], axis=(0, 1)) / nrows
        var = jnp.sum(hq_ref[...], axis=(0, 1)) / nrows - mu * mu
        a2 = g2_ref[0] * jax.lax.rsqrt(var + BN_EPS)
        c2 = be2_ref[0] - mu * a2
        g = jnp.maximum(h_ref[...] * a2[None, :] + c2[None, :], 0.0)
        g_buf[...] = g.astype(jnp.bfloat16)

    acc[...] += jax.lax.dot_general(
        sae_ref[...].astype(jnp.bfloat16), w_ref[...].astype(jnp.bfloat16),
        (((1,), (1,)), ((), ())), preferred_element_type=jnp.float32)

    @pl.when(k == nk - 1)
    def _():
        inter = jax.lax.dot_general(
            g_buf[...], w2_ref[...].astype(jnp.bfloat16),
            (((1,), (1,)), ((), ())),
            preferred_element_type=jnp.float32) + b2_ref[...]
        lin = acc[...] + lb_ref[...]
        lin_ref[...] = lin
        int_ref[...] = inter
        out_ref[...] = gb_ref[...] + lin + inter


def _final(sae, lin_w, h, hs, hq, w2, g2, be2, b2, lb, gb, bb, bo, bk):
    b, f = sae.shape
    o = lin_w.shape[0]
    d = h.shape[1]
    nb, no, nk = b // bb, o // bo, f // bk
    import functools
    out_shape = jax.ShapeDtypeStruct((b, o), jnp.float32)
    return pl.pallas_call(
        functools.partial(_final_body, nrows=float(b)),
        grid=(nb, no, nk),
        in_specs=[
            pl.BlockSpec((bb, bk), lambda i, j, k: (i, k)),
            pl.BlockSpec((bo, bk), lambda i, j, k: (j, k)),
            pl.BlockSpec((bb, d), lambda i, j, k: (i, 0)),
            pl.BlockSpec(hs.shape, lambda i, j, k: (0, 0, 0)),
            pl.BlockSpec(hq.shape, lambda i, j, k: (0, 0, 0)),
            pl.BlockSpec((bo, d), lambda i, j, k: (j, 0)),
            pl.BlockSpec((1, d), lambda i, j, k: (0, 0)),
            pl.BlockSpec((1, d), lambda i, j, k: (0, 0)),
            pl.BlockSpec((1, bo), lambda i, j, k: (0, j)),
            pl.BlockSpec((1, bo), lambda i, j, k: (0, j)),
            pl.BlockSpec((1, bo), lambda i, j, k: (0, j)),
        ],
        out_specs=[
            pl.BlockSpec((bb, bo), lambda i, j, k: (i, j)),
            pl.BlockSpec((bb, bo), lambda i, j, k: (i, j)),
            pl.BlockSpec((bb, bo), lambda i, j, k: (i, j)),
        ],
        out_shape=[out_shape, out_shape, out_shape],
        scratch_shapes=[
            pltpu.VMEM((bb, bo), jnp.float32),
            pltpu.VMEM((bb, d), jnp.bfloat16),
        ],
        compiler_params=pltpu.CompilerParams(
            dimension_semantics=("parallel", "arbitrary", "arbitrary"),
            vmem_limit_bytes=56 * 1024 * 1024,
        ),
        name="linear_mlp2_fused",
    )(sae, lin_w, h, hs, hq, w2, g2, be2, b2, lb, gb)


def kernel(sae_features, emb, lin_w, lin_b, global_bias, bn1_gamma, bn1_beta,
           mlp_w1, mlp_b1, bn2_gamma, bn2_beta, mlp_w2, mlp_b2):
    b, f = sae_features.shape
    d = emb.shape[1]

    thr, idx = _topk_thresholds(sae_features, bt=128)
    bi, s1, q1 = _bi_interaction(sae_features, thr, idx, emb, bb=1024, bk=2048)
    h, hs, hq = _mlp1(bi, s1, q1, mlp_w1.T, mlp_b1.reshape(1, d),
                      bn1_gamma.reshape(1, d), bn1_beta.reshape(1, d), bb=256)
    out, lin, inter = _final(
        sae_features, lin_w, h, hs, hq, mlp_w2,
        bn2_gamma.reshape(1, d), bn2_beta.reshape(1, d),
        mlp_b2.reshape(1, -1), lin_b.reshape(1, -1), global_bias.reshape(1, -1),
        bb=1024, bo=1024, bk=1024)
    return out, lin, inter


# speedup vs baseline: 1.1767x; 1.1166x over previous
import jax
import jax.numpy as jnp
from jax.experimental import pallas as pl
from jax.experimental.pallas import tpu as pltpu

TOP_K = 20
BN_EPS = 1e-5


_DEPTH = 4
_NCLS = 256


def _lane_fold(x, op):
    while x.shape[1] > _NCLS:
        h = x.shape[1] // 2
        x = op(x[:, :h], x[:, h:])
    return x


def _count_rounds(vals, cnts, nd):
    bt = vals[0].shape[0]
    zero = jnp.zeros((bt, 1), jnp.float32)
    rc, thr, nkeep = zero, zero - 1.0, zero
    found = zero > 1.0
    suspect = found
    for r in range(TOP_K):
        mm = vals[0]
        for d in range(1, nd):
            mm = jnp.maximum(mm, vals[d])
        m = jnp.max(mm, axis=1, keepdims=True)
        eqs = [vals[d] == m for d in range(nd)]
        cs = jnp.where(eqs[0], cnts[0], 0.0)
        for d in range(1, nd):
            cs = cs + jnp.where(eqs[d], cnts[d], 0.0)
        c = jnp.sum(cs, axis=1, keepdims=True)
        found_now = jnp.logical_and(jnp.logical_not(found), rc + c >= TOP_K)
        thr = jnp.where(found_now, m, thr)
        nkeep = jnp.where(found_now, TOP_K - rc, nkeep)
        found = jnp.logical_or(found, found_now)
        rc = rc + c
        if nd > 1:
            deep = jnp.max(jnp.where(eqs[nd - 1], 1.0, 0.0), axis=1,
                           keepdims=True) > 0.0
            suspect = jnp.logical_or(suspect,
                                     jnp.logical_and(deep,
                                                     jnp.logical_not(found)))
        if r < TOP_K - 1:
            for d in range(nd):
                vals[d] = jnp.where(eqs[d], -1.0, vals[d])
    return thr, nkeep, suspect


def _thr_body(sae_ref, thr_ref, idx_ref, work, vals_s, cnts_s, thr_s, nk_s,
              rc_s, fd_s):
    bt, f = sae_ref.shape
    work[...] = sae_ref[...]

    def _depth_body(d, carry):
        w = work[...]
        m = _lane_fold(w, jnp.maximum)
        eq = w == jnp.tile(m, (1, f // _NCLS))
        cnt = _lane_fold(jnp.where(eq, 1.0, 0.0), jnp.add)
        vals_s[d] = m
        cnts_s[d] = jnp.where(m >= 0.0, cnt, 0.0)
        work[...] = jnp.where(eq, -1.0, w)
        return carry

    jax.lax.fori_loop(0, _DEPTH, _depth_body, 0)
    vals = [vals_s[d] for d in range(_DEPTH)]
    cnts = [cnts_s[d] for d in range(_DEPTH)]
    thr, nkeep, suspect = _count_rounds(vals, cnts, _DEPTH)
    thr_s[...] = thr
    nk_s[...] = nkeep

    @pl.when(jnp.max(jnp.where(suspect, 1.0, 0.0)) > 0.0)
    def _():
        work[...] = sae_ref[...]
        rc_s[...] = jnp.zeros((bt, 1), jnp.float32)
        fd_s[...] = jnp.zeros((bt, 1), jnp.float32)

        def _bf_round(_r, carry):
            wv = work[...]
            m = jnp.max(wv, axis=1, keepdims=True)
            eq = wv == m
            c = jnp.sum(jnp.where(eq, 1.0, 0.0), axis=1, keepdims=True)
            rc = rc_s[...]
            fn = jnp.where(jnp.logical_and(fd_s[...] < 0.5, rc + c >= TOP_K),
                           1.0, 0.0)
            upd = jnp.where(suspect, fn, 0.0) > 0.5
            thr_s[...] = jnp.where(upd, m, thr_s[...])
            nk_s[...] = jnp.where(upd, TOP_K - rc, nk_s[...])
            fd_s[...] = jnp.maximum(fd_s[...], fn)
            rc_s[...] = rc + c
            work[...] = jnp.where(eq, -1.0, wv)
            return carry

        jax.lax.fori_loop(0, TOP_K, _bf_round, 0)

    iota_f = jax.lax.broadcasted_iota(jnp.int32, (bt, f), 1).astype(jnp.float32)
    thr_v = thr_s[...]
    work[...] = jnp.where(sae_ref[...] == thr_v, iota_f, jnp.inf)
    nk_s[...] = nk_s[...] - 1.0

    def _tie_round(_i, carry):
        @pl.when(jnp.max(nk_s[...]) > 0.0)
        def _():
            wv = work[...]
            mn = jnp.min(wv, axis=1, keepdims=True)
            rem = nk_s[...] > 0.0
            work[...] = jnp.where(jnp.logical_and(rem, wv == mn), jnp.inf, wv)
            nk_s[...] = jnp.where(rem, nk_s[...] - 1.0, nk_s[...])
        return carry

    jax.lax.fori_loop(0, TOP_K - 1, _tie_round, 0)
    thr_ref[...] = thr_v
    idx_ref[...] = jnp.min(work[...], axis=1, keepdims=True).astype(jnp.int32)


def _topk_thresholds(sae, bt):
    b, f = sae.shape
    return pl.pallas_call(
        _thr_body,
        grid=(b // bt,),
        in_specs=[pl.BlockSpec((bt, f), lambda i: (i, 0))],
        out_specs=[pl.BlockSpec((bt, 1), lambda i: (i, 0)),
                   pl.BlockSpec((bt, 1), lambda i: (i, 0))],
        out_shape=[jax.ShapeDtypeStruct((b, 1), jnp.float32),
                   jax.ShapeDtypeStruct((b, 1), jnp.int32)],
        scratch_shapes=[
            pltpu.VMEM((bt, f), jnp.float32),
            pltpu.VMEM((_DEPTH, bt, _NCLS), jnp.float32),
            pltpu.VMEM((_DEPTH, bt, _NCLS), jnp.float32),
            pltpu.VMEM((bt, 1), jnp.float32),
            pltpu.VMEM((bt, 1), jnp.float32),
            pltpu.VMEM((bt, 1), jnp.float32),
            pltpu.VMEM((bt, 1), jnp.float32),
        ],
        compiler_params=pltpu.CompilerParams(
            dimension_semantics=("parallel",),
            vmem_limit_bytes=48 * 1024 * 1024,
        ),
        name="topk_thr",
    )(sae)


def _bi_body(sae_ref, thr_ref, idx_ref, emb_ref, bi_ref, s_ref, q_ref,
             acc1, acc2, bk):
    k = pl.program_id(1)
    nk = pl.num_programs(1)

    @pl.when(k == 0)
    def _():
        acc1[...] = jnp.zeros_like(acc1)
        acc2[...] = jnp.zeros_like(acc2)

    blk = sae_ref[...]
    thr = thr_ref[...]
    g_iota = jax.lax.broadcasted_iota(jnp.int32, blk.shape, 1) + k * bk
    keep = (blk > thr) | ((blk == thr) & (g_iota <= idx_ref[...]))
    x = jnp.where(keep, blk, 0.0)
    e = emb_ref[...]
    acc1[...] += jnp.dot(x.astype(jnp.bfloat16), e.astype(jnp.bfloat16),
                         preferred_element_type=jnp.float32)
    acc2[...] += jnp.dot((x * x).astype(jnp.bfloat16),
                         (e * e).astype(jnp.bfloat16),
                         preferred_element_type=jnp.float32)

    @pl.when(k == nk - 1)
    def _():
        s = acc1[...]
        bi = 0.5 * (s * s - acc2[...])
        bi_ref[...] = bi
        s_ref[...] = jnp.sum(bi, axis=0, keepdims=True)[None]
        q_ref[...] = jnp.sum(bi * bi, axis=0, keepdims=True)[None]


def _bi_interaction(sae, thr, idx, emb, bb, bk):
    import functools
    b, f = sae.shape
    d = emb.shape[1]
    nb, nk = b // bb, f // bk
    return pl.pallas_call(
        functools.partial(_bi_body, bk=bk),
        grid=(nb, nk),
        in_specs=[
            pl.BlockSpec((bb, bk), lambda i, k: (i, k)),
            pl.BlockSpec((bb, 1), lambda i, k: (i, 0)),
            pl.BlockSpec((bb, 1), lambda i, k: (i, 0)),
            pl.BlockSpec((bk, d), lambda i, k: (k, 0)),
        ],
        out_specs=[
            pl.BlockSpec((bb, d), lambda i, k: (i, 0)),
            pl.BlockSpec((1, 1, d), lambda i, k: (i, 0, 0)),
            pl.BlockSpec((1, 1, d), lambda i, k: (i, 0, 0)),
        ],
        out_shape=[
            jax.ShapeDtypeStruct((b, d), jnp.float32),
            jax.ShapeDtypeStruct((nb, 1, d), jnp.float32),
            jax.ShapeDtypeStruct((nb, 1, d), jnp.float32),
        ],
        scratch_shapes=[
            pltpu.VMEM((bb, d), jnp.float32),
            pltpu.VMEM((bb, d), jnp.float32),
        ],
        compiler_params=pltpu.CompilerParams(
            dimension_semantics=("parallel", "arbitrary"),
            vmem_limit_bytes=48 * 1024 * 1024,
        ),
        name="bi_pool",
    )(sae, thr, idx, emb)


def _mlp1_body(bi_ref, s1_ref, q1_ref, w1t_ref, b1_ref, g1_ref, be1_ref,
               h_ref, hs_ref, hq_ref, nrows):
    mu = jnp.sum(s1_ref[...], axis=(0, 1)) / nrows
    var = jnp.sum(q1_ref[...], axis=(0, 1)) / nrows - mu * mu
    a1 = g1_ref[0] * jax.lax.rsqrt(var + BN_EPS)
    c1 = be1_ref[0] - mu * a1
    bi_n = bi_ref[...] * a1[None, :] + c1[None, :]
    h = jnp.dot(bi_n.astype(jnp.bfloat16), w1t_ref[...].astype(jnp.bfloat16),
                preferred_element_type=jnp.float32) + b1_ref[...]
    h_ref[...] = h
    hs_ref[...] = jnp.sum(h, axis=0, keepdims=True)[None]
    hq_ref[...] = jnp.sum(h * h, axis=0, keepdims=True)[None]


def _mlp1(bi, s1, q1, w1t, b1, g1, be1, bb):
    b, d = bi.shape
    nb = b // bb
    import functools
    return pl.pallas_call(
        functools.partial(_mlp1_body, nrows=float(b)),
        grid=(nb,),
        in_specs=[
            pl.BlockSpec((bb, d), lambda i: (i, 0)),
            pl.BlockSpec(s1.shape, lambda i: (0, 0, 0)),
            pl.BlockSpec(q1.shape, lambda i: (0, 0, 0)),
            pl.BlockSpec((d, d), lambda i: (0, 0)),
            pl.BlockSpec((1, d), lambda i: (0, 0)),
            pl.BlockSpec((1, d), lambda i: (0, 0)),
            pl.BlockSpec((1, d), lambda i: (0, 0)),
        ],
        out_specs=[
            pl.BlockSpec((bb, d), lambda i: (i, 0)),
            pl.BlockSpec((1, 1, d), lambda i: (i, 0, 0)),
            pl.BlockSpec((1, 1, d), lambda i: (i, 0, 0)),
        ],
        out_shape=[
            jax.ShapeDtypeStruct((b, d), jnp.float32),
            jax.ShapeDtypeStruct((nb, 1, d), jnp.float32),
            jax.ShapeDtypeStruct((nb, 1, d), jnp.float32),
        ],
        compiler_params=pltpu.CompilerParams(
            dimension_semantics=("parallel",),
            vmem_limit_bytes=40 * 1024 * 1024,
        ),
        name="mlp1_bn",
    )(bi, s1, q1, w1t, b1, g1, be1)


def _final_body(sae_ref, w_ref, h_ref, hs_ref, hq_ref, w2_ref, g2_ref,
                be2_ref, b2_ref, lb_ref, gb_ref,
                out_ref, lin_ref, int_ref, acc, g_buf, nrows):
    o = pl.program_id(1)
    k = pl.program_id(2)
    nk = pl.num_programs(2)

    @pl.when(k == 0)
    def _():
        acc[...] = jnp.zeros_like(acc)

    @pl.when((o == 0) & (k == 0))
    def _():
        mu = jnp.sum(hs_ref[...], axis=(0, 1)) / nrows
        var = jnp.sum(hq_ref[...], axis=(0, 1)) / nrows - mu * mu
        a2 = g2_ref[0] * jax.lax.rsqrt(var + BN_EPS)
        c2 = be2_ref[0] - mu * a2
        g = jnp.maximum(h_ref[...] * a2[None, :] + c2[None, :], 0.0)
        g_buf[...] = g.astype(jnp.bfloat16)

    acc[...] += jax.lax.dot_general(
        sae_ref[...].astype(jnp.bfloat16), w_ref[...].astype(jnp.bfloat16),
        (((1,), (1,)), ((), ())), preferred_element_type=jnp.float32)

    @pl.when(k == nk - 1)
    def _():
        inter = jax.lax.dot_general(
            g_buf[...], w2_ref[...].astype(jnp.bfloat16),
            (((1,), (1,)), ((), ())),
            preferred_element_type=jnp.float32) + b2_ref[...]
        lin = acc[...] + lb_ref[...]
        lin_ref[...] = lin
        int_ref[...] = inter
        out_ref[...] = gb_ref[...] + lin + inter


def _final(sae, lin_w, h, hs, hq, w2, g2, be2, b2, lb, gb, bb, bo, bk):
    b, f = sae.shape
    o = lin_w.shape[0]
    d = h.shape[1]
    nb, no, nk = b // bb, o // bo, f // bk
    import functools
    out_shape = jax.ShapeDtypeStruct((b, o), jnp.float32)
    return pl.pallas_call(
        functools.partial(_final_body, nrows=float(b)),
        grid=(nb, no, nk),
        in_specs=[
            pl.BlockSpec((bb, bk), lambda i, j, k: (i, k)),
            pl.BlockSpec((bo, bk), lambda i, j, k: (j, k)),
            pl.BlockSpec((bb, d), lambda i, j, k: (i, 0)),
            pl.BlockSpec(hs.shape, lambda i, j, k: (0, 0, 0)),
            pl.BlockSpec(hq.shape, lambda i, j, k: (0, 0, 0)),
            pl.BlockSpec((bo, d), lambda i, j, k: (j, 0)),
            pl.BlockSpec((1, d), lambda i, j, k: (0, 0)),
            pl.BlockSpec((1, d), lambda i, j, k: (0, 0)),
            pl.BlockSpec((1, bo), lambda i, j, k: (0, j)),
            pl.BlockSpec((1, bo), lambda i, j, k: (0, j)),
            pl.BlockSpec((1, bo), lambda i, j, k: (0, j)),
        ],
        out_specs=[
            pl.BlockSpec((bb, bo), lambda i, j, k: (i, j)),
            pl.BlockSpec((bb, bo), lambda i, j, k: (i, j)),
            pl.BlockSpec((bb, bo), lambda i, j, k: (i, j)),
        ],
        out_shape=[out_shape, out_shape, out_shape],
        scratch_shapes=[
            pltpu.VMEM((bb, bo), jnp.float32),
            pltpu.VMEM((bb, d), jnp.bfloat16),
        ],
        compiler_params=pltpu.CompilerParams(
            dimension_semantics=("parallel", "arbitrary", "arbitrary"),
            vmem_limit_bytes=56 * 1024 * 1024,
        ),
        name="linear_mlp2_fused",
    )(sae, lin_w, h, hs, hq, w2, g2, be2, b2, lb, gb)


def kernel(sae_features, emb, lin_w, lin_b, global_bias, bn1_gamma, bn1_beta,
           mlp_w1, mlp_b1, bn2_gamma, bn2_beta, mlp_w2, mlp_b2):
    b, f = sae_features.shape
    d = emb.shape[1]

    thr, idx = _topk_thresholds(sae_features, bt=128)
    bi, s1, q1 = _bi_interaction(sae_features, thr, idx, emb, bb=1024, bk=2048)
    h, hs, hq = _mlp1(bi, s1, q1, mlp_w1.T, mlp_b1.reshape(1, d),
                      bn1_gamma.reshape(1, d), bn1_beta.reshape(1, d), bb=256)
    out, lin, inter = _final(
        sae_features, lin_w, h, hs, hq, mlp_w2,
        bn2_gamma.reshape(1, d), bn2_beta.reshape(1, d),
        mlp_b2.reshape(1, -1), lin_b.reshape(1, -1), global_bias.reshape(1, -1),
        bb=1024, bo=1024, bk=1024)
    return out, lin, inter


# speedup vs baseline: 1.2338x; 1.0485x over previous
import jax
import jax.numpy as jnp
from jax.experimental import pallas as pl
from jax.experimental.pallas import tpu as pltpu

TOP_K = 20
BN_EPS = 1e-5


_DEPTH = 4
_NCLS = 256


def _lane_fold(x, op):
    while x.shape[1] > _NCLS:
        h = x.shape[1] // 2
        x = op(x[:, :h], x[:, h:])
    return x


def _count_rounds(vals, cnts, nd):
    bt = vals[0].shape[0]
    zero = jnp.zeros((bt, 1), jnp.float32)
    rc, thr, nkeep = zero, zero - 1.0, zero
    found = zero > 1.0
    suspect = found
    for r in range(TOP_K):
        mm = vals[0]
        for d in range(1, nd):
            mm = jnp.maximum(mm, vals[d])
        m = jnp.max(mm, axis=1, keepdims=True)
        eqs = [vals[d] == m for d in range(nd)]
        cs = jnp.where(eqs[0], cnts[0], 0.0)
        for d in range(1, nd):
            cs = cs + jnp.where(eqs[d], cnts[d], 0.0)
        c = jnp.sum(cs, axis=1, keepdims=True)
        found_now = jnp.logical_and(jnp.logical_not(found), rc + c >= TOP_K)
        thr = jnp.where(found_now, m, thr)
        nkeep = jnp.where(found_now, TOP_K - rc, nkeep)
        found = jnp.logical_or(found, found_now)
        rc = rc + c
        if nd > 1:
            deep = jnp.max(jnp.where(eqs[nd - 1], 1.0, 0.0), axis=1,
                           keepdims=True) > 0.0
            suspect = jnp.logical_or(suspect,
                                     jnp.logical_and(deep,
                                                     jnp.logical_not(found)))
        if r < TOP_K - 1:
            for d in range(nd):
                vals[d] = jnp.where(eqs[d], -1.0, vals[d])
    return thr, nkeep, suspect


def _thr_body(sae_ref, thr_ref, idx_ref, sae_bf_ref, work, vals_s, cnts_s,
              thr_s, nk_s, rc_s, fd_s):
    bt, f = sae_ref.shape
    sae_bf_ref[...] = sae_ref[...].astype(jnp.bfloat16)
    work[...] = sae_ref[...]

    def _depth_body(d, carry):
        w = work[...]
        m = _lane_fold(w, jnp.maximum)
        eq = w == jnp.tile(m, (1, f // _NCLS))
        cnt = _lane_fold(jnp.where(eq, 1.0, 0.0), jnp.add)
        vals_s[d] = m
        cnts_s[d] = jnp.where(m >= 0.0, cnt, 0.0)
        work[...] = jnp.where(eq, -1.0, w)
        return carry

    jax.lax.fori_loop(0, _DEPTH, _depth_body, 0)
    vals = [vals_s[d] for d in range(_DEPTH)]
    cnts = [cnts_s[d] for d in range(_DEPTH)]
    thr, nkeep, suspect = _count_rounds(vals, cnts, _DEPTH)
    thr_s[...] = thr
    nk_s[...] = nkeep

    @pl.when(jnp.max(jnp.where(suspect, 1.0, 0.0)) > 0.0)
    def _():
        work[...] = sae_ref[...]
        rc_s[...] = jnp.zeros((bt, 1), jnp.float32)
        fd_s[...] = jnp.zeros((bt, 1), jnp.float32)

        def _bf_round(_r, carry):
            wv = work[...]
            m = jnp.max(wv, axis=1, keepdims=True)
            eq = wv == m
            c = jnp.sum(jnp.where(eq, 1.0, 0.0), axis=1, keepdims=True)
            rc = rc_s[...]
            fn = jnp.where(jnp.logical_and(fd_s[...] < 0.5, rc + c >= TOP_K),
                           1.0, 0.0)
            upd = jnp.where(suspect, fn, 0.0) > 0.5
            thr_s[...] = jnp.where(upd, m, thr_s[...])
            nk_s[...] = jnp.where(upd, TOP_K - rc, nk_s[...])
            fd_s[...] = jnp.maximum(fd_s[...], fn)
            rc_s[...] = rc + c
            work[...] = jnp.where(eq, -1.0, wv)
            return carry

        jax.lax.fori_loop(0, TOP_K, _bf_round, 0)

    iota_f = jax.lax.broadcasted_iota(jnp.int32, (bt, f), 1).astype(jnp.float32)
    thr_v = thr_s[...]
    work[...] = jnp.where(sae_ref[...] == thr_v, iota_f, jnp.inf)
    nk_s[...] = nk_s[...] - 1.0

    def _tie_round(_i, carry):
        @pl.when(jnp.max(nk_s[...]) > 0.0)
        def _():
            wv = work[...]
            mn = jnp.min(wv, axis=1, keepdims=True)
            rem = nk_s[...] > 0.0
            work[...] = jnp.where(jnp.logical_and(rem, wv == mn), jnp.inf, wv)
            nk_s[...] = jnp.where(rem, nk_s[...] - 1.0, nk_s[...])
        return carry

    jax.lax.fori_loop(0, TOP_K - 1, _tie_round, 0)
    thr_ref[...] = thr_v
    idx_ref[...] = jnp.min(work[...], axis=1, keepdims=True).astype(jnp.int32)


def _topk_thresholds(sae, bt):
    b, f = sae.shape
    return pl.pallas_call(
        _thr_body,
        grid=(b // bt,),
        in_specs=[pl.BlockSpec((bt, f), lambda i: (i, 0))],
        out_specs=[pl.BlockSpec((bt, 1), lambda i: (i, 0)),
                   pl.BlockSpec((bt, 1), lambda i: (i, 0)),
                   pl.BlockSpec((bt, f), lambda i: (i, 0))],
        out_shape=[jax.ShapeDtypeStruct((b, 1), jnp.float32),
                   jax.ShapeDtypeStruct((b, 1), jnp.int32),
                   jax.ShapeDtypeStruct((b, f), jnp.bfloat16)],
        scratch_shapes=[
            pltpu.VMEM((bt, f), jnp.float32),
            pltpu.VMEM((_DEPTH, bt, _NCLS), jnp.float32),
            pltpu.VMEM((_DEPTH, bt, _NCLS), jnp.float32),
            pltpu.VMEM((bt, 1), jnp.float32),
            pltpu.VMEM((bt, 1), jnp.float32),
            pltpu.VMEM((bt, 1), jnp.float32),
            pltpu.VMEM((bt, 1), jnp.float32),
        ],
        compiler_params=pltpu.CompilerParams(
            dimension_semantics=("parallel",),
            vmem_limit_bytes=48 * 1024 * 1024,
        ),
        name="topk_thr",
    )(sae)


def _bi_body(sae_ref, thr_ref, idx_ref, emb_ref, bi_ref, s_ref, q_ref,
             acc1, acc2, bk):
    k = pl.program_id(1)
    nk = pl.num_programs(1)

    @pl.when(k == 0)
    def _():
        acc1[...] = jnp.zeros_like(acc1)
        acc2[...] = jnp.zeros_like(acc2)

    blk = sae_ref[...]
    thr = thr_ref[...]
    g_iota = jax.lax.broadcasted_iota(jnp.int32, blk.shape, 1) + k * bk
    keep = (blk > thr) | ((blk == thr) & (g_iota <= idx_ref[...]))
    x = jnp.where(keep, blk, 0.0)
    e = emb_ref[...]
    acc1[...] += jnp.dot(x.astype(jnp.bfloat16), e.astype(jnp.bfloat16),
                         preferred_element_type=jnp.float32)
    acc2[...] += jnp.dot((x * x).astype(jnp.bfloat16),
                         (e * e).astype(jnp.bfloat16),
                         preferred_element_type=jnp.float32)

    @pl.when(k == nk - 1)
    def _():
        s = acc1[...]
        bi = 0.5 * (s * s - acc2[...])
        bi_ref[...] = bi
        s_ref[...] = jnp.sum(bi, axis=0, keepdims=True)[None]
        q_ref[...] = jnp.sum(bi * bi, axis=0, keepdims=True)[None]


def _bi_interaction(sae, thr, idx, emb, bb, bk):
    import functools
    b, f = sae.shape
    d = emb.shape[1]
    nb, nk = b // bb, f // bk
    return pl.pallas_call(
        functools.partial(_bi_body, bk=bk),
        grid=(nb, nk),
        in_specs=[
            pl.BlockSpec((bb, bk), lambda i, k: (i, k)),
            pl.BlockSpec((bb, 1), lambda i, k: (i, 0)),
            pl.BlockSpec((bb, 1), lambda i, k: (i, 0)),
            pl.BlockSpec((bk, d), lambda i, k: (k, 0)),
        ],
        out_specs=[
            pl.BlockSpec((bb, d), lambda i, k: (i, 0)),
            pl.BlockSpec((1, 1, d), lambda i, k: (i, 0, 0)),
            pl.BlockSpec((1, 1, d), lambda i, k: (i, 0, 0)),
        ],
        out_shape=[
            jax.ShapeDtypeStruct((b, d), jnp.float32),
            jax.ShapeDtypeStruct((nb, 1, d), jnp.float32),
            jax.ShapeDtypeStruct((nb, 1, d), jnp.float32),
        ],
        scratch_shapes=[
            pltpu.VMEM((bb, d), jnp.float32),
            pltpu.VMEM((bb, d), jnp.float32),
        ],
        compiler_params=pltpu.CompilerParams(
            dimension_semantics=("parallel", "arbitrary"),
            vmem_limit_bytes=48 * 1024 * 1024,
        ),
        name="bi_pool",
    )(sae, thr, idx, emb)


def _mlp1_body(bi_ref, s1_ref, q1_ref, w1t_ref, b1_ref, g1_ref, be1_ref,
               h_ref, hs_ref, hq_ref, nrows):
    mu = jnp.sum(s1_ref[...], axis=(0, 1)) / nrows
    var = jnp.sum(q1_ref[...], axis=(0, 1)) / nrows - mu * mu
    a1 = g1_ref[0] * jax.lax.rsqrt(var + BN_EPS)
    c1 = be1_ref[0] - mu * a1
    bi_n = bi_ref[...] * a1[None, :] + c1[None, :]
    h = jnp.dot(bi_n.astype(jnp.bfloat16), w1t_ref[...].astype(jnp.bfloat16),
                preferred_element_type=jnp.float32) + b1_ref[...]
    h_ref[...] = h
    hs_ref[...] = jnp.sum(h, axis=0, keepdims=True)[None]
    hq_ref[...] = jnp.sum(h * h, axis=0, keepdims=True)[None]


def _mlp1(bi, s1, q1, w1t, b1, g1, be1, bb):
    b, d = bi.shape
    nb = b // bb
    import functools
    return pl.pallas_call(
        functools.partial(_mlp1_body, nrows=float(b)),
        grid=(nb,),
        in_specs=[
            pl.BlockSpec((bb, d), lambda i: (i, 0)),
            pl.BlockSpec(s1.shape, lambda i: (0, 0, 0)),
            pl.BlockSpec(q1.shape, lambda i: (0, 0, 0)),
            pl.BlockSpec((d, d), lambda i: (0, 0)),
            pl.BlockSpec((1, d), lambda i: (0, 0)),
            pl.BlockSpec((1, d), lambda i: (0, 0)),
            pl.BlockSpec((1, d), lambda i: (0, 0)),
        ],
        out_specs=[
            pl.BlockSpec((bb, d), lambda i: (i, 0)),
            pl.BlockSpec((1, 1, d), lambda i: (i, 0, 0)),
            pl.BlockSpec((1, 1, d), lambda i: (i, 0, 0)),
        ],
        out_shape=[
            jax.ShapeDtypeStruct((b, d), jnp.float32),
            jax.ShapeDtypeStruct((nb, 1, d), jnp.float32),
            jax.ShapeDtypeStruct((nb, 1, d), jnp.float32),
        ],
        compiler_params=pltpu.CompilerParams(
            dimension_semantics=("parallel",),
            vmem_limit_bytes=40 * 1024 * 1024,
        ),
        name="mlp1_bn",
    )(bi, s1, q1, w1t, b1, g1, be1)


def _final_body(sae_ref, w_ref, h_ref, hs_ref, hq_ref, w2_ref, g2_ref,
                be2_ref, b2_ref, lb_ref, gb_ref,
                out_ref, lin_ref, int_ref, acc, g_buf, nrows):
    o = pl.program_id(1)
    k = pl.program_id(2)
    nk = pl.num_programs(2)

    @pl.when(k == 0)
    def _():
        acc[...] = jnp.zeros_like(acc)

    @pl.when((o == 0) & (k == 0))
    def _():
        mu = jnp.sum(hs_ref[...], axis=(0, 1)) / nrows
        var = jnp.sum(hq_ref[...], axis=(0, 1)) / nrows - mu * mu
        a2 = g2_ref[0] * jax.lax.rsqrt(var + BN_EPS)
        c2 = be2_ref[0] - mu * a2
        g = jnp.maximum(h_ref[...] * a2[None, :] + c2[None, :], 0.0)
        g_buf[...] = g.astype(jnp.bfloat16)

    acc[...] += jax.lax.dot_general(
        sae_ref[...], w_ref[...].astype(jnp.bfloat16),
        (((1,), (1,)), ((), ())), preferred_element_type=jnp.float32)

    @pl.when(k == nk - 1)
    def _():
        inter = jax.lax.dot_general(
            g_buf[...], w2_ref[...].astype(jnp.bfloat16),
            (((1,), (1,)), ((), ())),
            preferred_element_type=jnp.float32) + b2_ref[...]
        lin = acc[...] + lb_ref[...]
        lin_ref[...] = lin
        int_ref[...] = inter
        out_ref[...] = gb_ref[...] + lin + inter


def _final(sae, lin_w, h, hs, hq, w2, g2, be2, b2, lb, gb, bb, bo, bk):
    b, f = sae.shape
    o = lin_w.shape[0]
    d = h.shape[1]
    nb, no, nk = b // bb, o // bo, f // bk
    import functools
    out_shape = jax.ShapeDtypeStruct((b, o), jnp.float32)
    return pl.pallas_call(
        functools.partial(_final_body, nrows=float(b)),
        grid=(nb, no, nk),
        in_specs=[
            pl.BlockSpec((bb, bk), lambda i, j, k: (i, k)),
            pl.BlockSpec((bo, bk), lambda i, j, k: (j, k)),
            pl.BlockSpec((bb, d), lambda i, j, k: (i, 0)),
            pl.BlockSpec(hs.shape, lambda i, j, k: (0, 0, 0)),
            pl.BlockSpec(hq.shape, lambda i, j, k: (0, 0, 0)),
            pl.BlockSpec((bo, d), lambda i, j, k: (j, 0)),
            pl.BlockSpec((1, d), lambda i, j, k: (0, 0)),
            pl.BlockSpec((1, d), lambda i, j, k: (0, 0)),
            pl.BlockSpec((1, bo), lambda i, j, k: (0, j)),
            pl.BlockSpec((1, bo), lambda i, j, k: (0, j)),
            pl.BlockSpec((1, bo), lambda i, j, k: (0, j)),
        ],
        out_specs=[
            pl.BlockSpec((bb, bo), lambda i, j, k: (i, j)),
            pl.BlockSpec((bb, bo), lambda i, j, k: (i, j)),
            pl.BlockSpec((bb, bo), lambda i, j, k: (i, j)),
        ],
        out_shape=[out_shape, out_shape, out_shape],
        scratch_shapes=[
            pltpu.VMEM((bb, bo), jnp.float32),
            pltpu.VMEM((bb, d), jnp.bfloat16),
        ],
        compiler_params=pltpu.CompilerParams(
            dimension_semantics=("parallel", "arbitrary", "arbitrary"),
            vmem_limit_bytes=56 * 1024 * 1024,
        ),
        name="linear_mlp2_fused",
    )(sae, lin_w, h, hs, hq, w2, g2, be2, b2, lb, gb)


def kernel(sae_features, emb, lin_w, lin_b, global_bias, bn1_gamma, bn1_beta,
           mlp_w1, mlp_b1, bn2_gamma, bn2_beta, mlp_w2, mlp_b2):
    b, f = sae_features.shape
    d = emb.shape[1]

    thr, idx, sae_bf = _topk_thresholds(sae_features, bt=128)
    bi, s1, q1 = _bi_interaction(sae_features, thr, idx, emb, bb=1024, bk=2048)
    h, hs, hq = _mlp1(bi, s1, q1, mlp_w1.T, mlp_b1.reshape(1, d),
                      bn1_gamma.reshape(1, d), bn1_beta.reshape(1, d), bb=256)
    out, lin, inter = _final(
        sae_bf, lin_w, h, hs, hq, mlp_w2,
        bn2_gamma.reshape(1, d), bn2_beta.reshape(1, d),
        mlp_b2.reshape(1, -1), lin_b.reshape(1, -1), global_bias.reshape(1, -1),
        bb=1024, bo=1024, bk=1024)
    return out, lin, inter
